# baseline (speedup 1.0000x reference)
import functools

import jax
import jax.numpy as jnp
from jax import lax
from jax.experimental import pallas as pl
from jax.experimental.pallas import tpu as pltpu
from jax.experimental.pallas import tpu_sc as plsc

B, S, H = 1, 2048, 1024
I = 512
E = 8
NGROUP = 4
GSIZE = E // NGROUP
SCALE = 1.0
TOPK = 2

TM = 256
NT2 = (S * TOPK) // TM + E
NSLOT = NT2 * TM
HPK = H // 2

NW = 32
TPW = S // NW
CH = 8


def _shift_down(c, k):
    return jnp.concatenate([jnp.zeros((k, E), c.dtype), c[:S - k]], axis=0)


def _gate_dispatch_body(x_ref, gw_ref, b_ref,
                        dsel_ref, wsel_ref, eid_ref, nuse_ref):
    x = x_ref[...]
    gw = gw_ref[...]
    logits = jax.lax.dot_general(
        x, gw, (((1,), (1,)), ((), ())),
        preferred_element_type=jnp.float32)
    scores = jax.nn.sigmoid(logits)
    s4c = scores + b_ref[...]

    gcols = [s4c[:, 2 * g:2 * g + 1] + s4c[:, 2 * g + 1:2 * g + 2]
             for g in range(NGROUP)]
    gexp = jnp.concatenate(
        [gcols[g] for g in range(NGROUP) for _ in range(GSIZE)], axis=1)

    eids = lax.broadcasted_iota(jnp.int32, (1, E), 1)
    gids = eids // GSIZE

    cnt = jnp.zeros((S, E), jnp.int32)
    for gp in range(NGROUP):
        gsp = gcols[gp]
        beats = (gsp > gexp) | ((gsp == gexp) & (gp < gids))
        cnt = cnt + beats.astype(jnp.int32)
    gmask = cnt < 2

    tmp = jnp.where(gmask, s4c, 0.0)
    cnt2 = jnp.zeros((S, E), jnp.int32)
    for ep in range(E):
        v = tmp[:, ep:ep + 1]
        beats = (v > tmp) | ((v == tmp) & (ep < eids))
        cnt2 = cnt2 + beats.astype(jnp.int32)
    sel = cnt2 < 2

    w = jnp.where(sel, scores, 0.0)
    denom = jnp.sum(w, axis=1, keepdims=True) + 1e-20
    wn = w / denom * SCALE

    seli = sel.astype(jnp.int32)
    c = seli
    k = 1
    while k < S:
        c = c + _shift_down(c, k)
        k *= 2
    pos = c - seli
    cnt_tot = c[S - 1:S, :]
    pc = ((cnt_tot + (TM - 1)) // TM) * TM

    r8 = lax.broadcasted_iota(jnp.int32, (E, E), 0)
    c8 = lax.broadcasted_iota(jnp.int32, (E, E), 1)
    ltf = (r8 < c8).astype(jnp.float32)
    off = jax.lax.dot_general(
        pc.astype(jnp.float32), ltf, (((1,), (0,)), ((), ())),
        preferred_element_type=jnp.float32).astype(jnp.int32)
    dest = off + pos

    lc = jax.lax.dot_general(
        sel.astype(jnp.float32), ltf, (((1,), (0,)), ((), ())),
        preferred_element_type=jnp.float32).astype(jnp.int32)
    first = sel & (lc == 0)
    second = sel & (lc == 1)
    d0 = jnp.sum(jnp.where(first, dest, 0), axis=1, keepdims=True)
    d1 = jnp.sum(jnp.where(second, dest, 0), axis=1, keepdims=True)
    w0 = jnp.sum(jnp.where(first, wn, 0.0), axis=1, keepdims=True)
    w1 = jnp.sum(jnp.where(second, wn, 0.0), axis=1, keepdims=True)
    dsel_ref[...] = jnp.concatenate([d0, d1], axis=1)
    wsel_ref[...] = jnp.concatenate([w0, w1], axis=1)

    ti = lax.broadcasted_iota(jnp.int32, (1, NT2), 1) * TM
    ecnt = jnp.zeros((1, NT2), jnp.int32)
    for e in range(E):
        endv = off[:, e:e + 1] + pc[:, e:e + 1]
        ecnt = ecnt + (ti >= endv).astype(jnp.int32)
    eid_ref[...] = jnp.minimum(ecnt, E - 1)
    total = off[:, E - 1:E] + pc[:, E - 1:E]
    nuse_ref[...] = total // TM


def _wcast_body(gp_ref, up_ref, dp_ref, gpb_ref, upb_ref, dpb_ref):
    gpb_ref[...] = gp_ref[...].astype(jnp.bfloat16)
    upb_ref[...] = up_ref[...].astype(jnp.bfloat16)
    dpb_ref[...] = dp_ref[...].astype(jnp.bfloat16)


def _gmm_body(eid_ref, nuse_ref, xg_ref, gp_ref, up_ref, dp_ref, yp_ref):
    s = pl.program_id(0)

    @pl.when(s < nuse_ref[0])
    def _compute():
        x = xg_ref[...].astype(jnp.bfloat16)
        g = jax.lax.dot_general(x, gp_ref[0], (((1,), (1,)), ((), ())),
                                preferred_element_type=jnp.float32)
        u = jax.lax.dot_general(x, up_ref[0], (((1,), (1,)), ((), ())),
                                preferred_element_type=jnp.float32)
        hact = (g * jax.nn.sigmoid(g) * u).astype(jnp.bfloat16)
        yp_ref[...] = jax.lax.dot_general(
            hact, dp_ref[0], (((1,), (1,)), ((), ())),
            preferred_element_type=jnp.float32)


def _disp_body(xpk, d3, xg, rows_v, i0, i1, s0, s1):
    wid = lax.axis_index("s") * 2 + lax.axis_index("c")
    base = wid * TPW
    pltpu.sync_copy(xpk.at[pl.ds(base, TPW)], rows_v)
    pltpu.sync_copy(d3.at[2 * wid], i0)
    pltpu.sync_copy(d3.at[2 * wid + 1], i1)
    c0 = pltpu.async_copy(rows_v, xg.at[i0], s0)
    c1 = pltpu.async_copy(rows_v, xg.at[i1], s1)
    c0.wait()
    c1.wait()


def _comb_body(yp, d3, w16, out, i0, i1, w0, w1, y0, y1, ob, s0, s1):
    wid = lax.axis_index("s") * 2 + lax.axis_index("c")
    base = wid * TPW
    pltpu.sync_copy(d3.at[2 * wid], i0)
    pltpu.sync_copy(d3.at[2 * wid + 1], i1)
    pltpu.sync_copy(w16.at[2 * wid], w0)
    pltpu.sync_copy(w16.at[2 * wid + 1], w1)

    def chunk(c, _):
        ca = pltpu.async_copy(yp.at[i0.at[pl.ds(c * CH, CH)]], y0, s0)
        cb = pltpu.async_copy(yp.at[i1.at[pl.ds(c * CH, CH)]], y1, s1)
        ca.wait()
        cb.wait()
        for j in range(CH):
            w0j = w0[pl.ds((c * CH + j) * 16, 16)]
            w1j = w1[pl.ds((c * CH + j) * 16, 16)]
            for k in range(H // 16):
                a = y0[j, pl.ds(k * 16, 16)]
                b = y1[j, pl.ds(k * 16, 16)]
                ob[j, pl.ds(k * 16, 16)] = w0j * a + w1j * b
        pltpu.sync_copy(ob, out.at[pl.ds(base + c * CH, CH)])
        return ()

    lax.fori_loop(0, TPW // CH, chunk, (), unroll=False)


@functools.lru_cache(maxsize=1)
def _sc_kernels():
    mesh = plsc.VectorSubcoreMesh(core_axis_name="c", subcore_axis_name="s",
                                  num_cores=2, num_subcores=16)
    disp = pl.kernel(
        _disp_body,
        mesh=mesh,
        compiler_params=pltpu.CompilerParams(use_tc_tiling_on_sc=True),
        out_type=jax.ShapeDtypeStruct((NSLOT, H), jnp.float32),
        scratch_types=[
            pltpu.VMEM((TPW, H), jnp.float32),
            pltpu.VMEM((TPW,), jnp.int32),
            pltpu.VMEM((TPW,), jnp.int32),
            pltpu.SemaphoreType.DMA,
            pltpu.SemaphoreType.DMA,
        ],
    )
    comb = pl.kernel(
        _comb_body,
        mesh=mesh,
        compiler_params=pltpu.CompilerParams(use_tc_tiling_on_sc=True),
        out_type=jax.ShapeDtypeStruct((S, H), jnp.float32),
        scratch_types=[
            pltpu.VMEM((TPW,), jnp.int32),
            pltpu.VMEM((TPW,), jnp.int32),
            pltpu.VMEM((TPW * 16,), jnp.float32),
            pltpu.VMEM((TPW * 16,), jnp.float32),
            pltpu.VMEM((CH, H), jnp.float32),
            pltpu.VMEM((CH, H), jnp.float32),
            pltpu.VMEM((CH, H), jnp.float32),
            pltpu.SemaphoreType.DMA,
            pltpu.SemaphoreType.DMA,
        ],
    )
    return disp, comb


@jax.jit
def _run(x, gate_weight, bias2d, gate_proj, up_proj, down_proj):
    dsel, wsel, eid, nuse = pl.pallas_call(
        _gate_dispatch_body,
        out_shape=(
            jax.ShapeDtypeStruct((S, TOPK), jnp.int32),
            jax.ShapeDtypeStruct((S, TOPK), jnp.float32),
            jax.ShapeDtypeStruct((1, NT2), jnp.int32),
            jax.ShapeDtypeStruct((1, 1), jnp.int32),
        ),
    )(x, gate_weight, bias2d)

    d3 = dsel.reshape(NW, TPW, TOPK).transpose(0, 2, 1).reshape(
        NW * TOPK, TPW)
    w3 = wsel.reshape(NW, TPW, TOPK).transpose(0, 2, 1).reshape(
        NW * TOPK, TPW)

    disp, comb = _sc_kernels()
    xg = disp(x, d3)

    gpb, upb, dpb = pl.pallas_call(
        _wcast_body,
        grid=(E,),
        in_specs=[
            pl.BlockSpec((1, I, H), lambda e: (e, 0, 0)),
            pl.BlockSpec((1, I, H), lambda e: (e, 0, 0)),
            pl.BlockSpec((1, H, I), lambda e: (e, 0, 0)),
        ],
        out_specs=[
            pl.BlockSpec((1, I, H), lambda e: (e, 0, 0)),
            pl.BlockSpec((1, I, H), lambda e: (e, 0, 0)),
            pl.BlockSpec((1, H, I), lambda e: (e, 0, 0)),
        ],
        out_shape=(
            jax.ShapeDtypeStruct((E, I, H), jnp.bfloat16),
            jax.ShapeDtypeStruct((E, I, H), jnp.bfloat16),
            jax.ShapeDtypeStruct((E, H, I), jnp.bfloat16),
        ),
    )(gate_proj, up_proj, down_proj)

    grid_spec = pltpu.PrefetchScalarGridSpec(
        num_scalar_prefetch=2,
        grid=(NT2,),
        in_specs=[
            pl.BlockSpec((TM, H), lambda s, eid, nuse: (s, 0)),
            pl.BlockSpec((1, I, H), lambda s, eid, nuse: (eid[s], 0, 0)),
            pl.BlockSpec((1, I, H), lambda s, eid, nuse: (eid[s], 0, 0)),
            pl.BlockSpec((1, H, I), lambda s, eid, nuse: (eid[s], 0, 0)),
        ],
        out_specs=pl.BlockSpec((TM, H), lambda s, eid, nuse: (s, 0)),
    )
    yp = pl.pallas_call(
        _gmm_body,
        grid_spec=grid_spec,
        out_shape=jax.ShapeDtypeStruct((NSLOT, H), jnp.float32),
    )(eid.reshape(NT2), nuse.reshape(1), xg, gpb, upb, dpb)

    w16 = jnp.broadcast_to(w3[:, :, None], (NW * TOPK, TPW, 16)).reshape(
        NW * TOPK, TPW * 16)
    return comb(yp, d3, w16)


def kernel(hidden_states, gate_weight, e_score_correction_bias,
           gate_proj, up_proj, down_proj):
    x = hidden_states.reshape(-1, H).astype(jnp.float32)
    bias2d = e_score_correction_bias.reshape(1, E).astype(jnp.float32)
    out = _run(x, gate_weight, bias2d, gate_proj, up_proj, down_proj)
    return out.reshape(hidden_states.shape)


# speedup vs baseline: 1.1540x; 1.1540x over previous
import functools

import jax
import jax.numpy as jnp
from jax import lax
from jax.experimental import pallas as pl
from jax.experimental.pallas import tpu as pltpu
from jax.experimental.pallas import tpu_sc as plsc

B, S, H = 1, 2048, 1024
I = 512
E = 8
NGROUP = 4
GSIZE = E // NGROUP
SCALE = 1.0
TOPK = 2

TM = 256
NT2 = (S * TOPK) // TM + E
NSLOT = NT2 * TM
HPK = H // 2

NW = 32
TPW = S // NW
CH = 8


def _shift_down(c, k):
    return jnp.concatenate([jnp.zeros((k, E), c.dtype), c[:S - k]], axis=0)


def _gate_dispatch_body(x_ref, gw_ref, b_ref,
                        dsel_ref, wsel_ref, eid_ref, nuse_ref):
    x = x_ref[...]
    gw = gw_ref[...]
    logits = jax.lax.dot_general(
        x, gw, (((1,), (1,)), ((), ())),
        preferred_element_type=jnp.float32)
    scores = jax.nn.sigmoid(logits)
    s4c = scores + b_ref[...]

    gcols = [s4c[:, 2 * g:2 * g + 1] + s4c[:, 2 * g + 1:2 * g + 2]
             for g in range(NGROUP)]
    gexp = jnp.concatenate(
        [gcols[g] for g in range(NGROUP) for _ in range(GSIZE)], axis=1)

    eids = lax.broadcasted_iota(jnp.int32, (1, E), 1)
    gids = eids // GSIZE

    cnt = jnp.zeros((S, E), jnp.int32)
    for gp in range(NGROUP):
        gsp = gcols[gp]
        beats = (gsp > gexp) | ((gsp == gexp) & (gp < gids))
        cnt = cnt + beats.astype(jnp.int32)
    gmask = cnt < 2

    tmp = jnp.where(gmask, s4c, 0.0)
    cnt2 = jnp.zeros((S, E), jnp.int32)
    for ep in range(E):
        v = tmp[:, ep:ep + 1]
        beats = (v > tmp) | ((v == tmp) & (ep < eids))
        cnt2 = cnt2 + beats.astype(jnp.int32)
    sel = cnt2 < 2

    w = jnp.where(sel, scores, 0.0)
    denom = jnp.sum(w, axis=1, keepdims=True) + 1e-20
    wn = w / denom * SCALE

    seli = sel.astype(jnp.int32)
    c = seli
    k = 1
    while k < S:
        c = c + _shift_down(c, k)
        k *= 2
    pos = c - seli
    cnt_tot = c[S - 1:S, :]
    pc = ((cnt_tot + (TM - 1)) // TM) * TM

    r8 = lax.broadcasted_iota(jnp.int32, (E, E), 0)
    c8 = lax.broadcasted_iota(jnp.int32, (E, E), 1)
    ltf = (r8 < c8).astype(jnp.float32)
    off = jax.lax.dot_general(
        pc.astype(jnp.float32), ltf, (((1,), (0,)), ((), ())),
        preferred_element_type=jnp.float32).astype(jnp.int32)
    dest = off + pos

    lc = jax.lax.dot_general(
        sel.astype(jnp.float32), ltf, (((1,), (0,)), ((), ())),
        preferred_element_type=jnp.float32).astype(jnp.int32)
    first = sel & (lc == 0)
    second = sel & (lc == 1)
    d0 = jnp.sum(jnp.where(first, dest, 0), axis=1, keepdims=True)
    d1 = jnp.sum(jnp.where(second, dest, 0), axis=1, keepdims=True)
    w0 = jnp.sum(jnp.where(first, wn, 0.0), axis=1, keepdims=True)
    w1 = jnp.sum(jnp.where(second, wn, 0.0), axis=1, keepdims=True)
    dsel_ref[...] = jnp.concatenate([d0, d1], axis=1)
    wsel_ref[...] = jnp.concatenate([w0, w1], axis=1)

    ti = lax.broadcasted_iota(jnp.int32, (1, NT2), 1) * TM
    ecnt = jnp.zeros((1, NT2), jnp.int32)
    for e in range(E):
        endv = off[:, e:e + 1] + pc[:, e:e + 1]
        ecnt = ecnt + (ti >= endv).astype(jnp.int32)
    eid_ref[...] = jnp.minimum(ecnt, E - 1)
    total = off[:, E - 1:E] + pc[:, E - 1:E]
    nuse_ref[...] = total // TM


def _gmm_body(eid_ref, nuse_ref, xg_ref, gp_ref, up_ref, dp_ref, yp_ref,
              wg_s, wu_s, wd_s):
    s = pl.program_id(0)

    @pl.when(s < E)
    def _cast():
        wg_s[pl.ds(s, 1)] = gp_ref[...].astype(jnp.bfloat16)
        wu_s[pl.ds(s, 1)] = up_ref[...].astype(jnp.bfloat16)
        wd_s[pl.ds(s, 1)] = dp_ref[...].astype(jnp.bfloat16)

    @pl.when((s >= E) & (s - E < nuse_ref[0]))
    def _compute():
        e = eid_ref[s - E]
        x = xg_ref[...].astype(jnp.bfloat16)
        g = jax.lax.dot_general(x, wg_s[e], (((1,), (1,)), ((), ())),
                                preferred_element_type=jnp.float32)
        u = jax.lax.dot_general(x, wu_s[e], (((1,), (1,)), ((), ())),
                                preferred_element_type=jnp.float32)
        hact = (g * jax.nn.sigmoid(g) * u).astype(jnp.bfloat16)
        yp_ref[...] = jax.lax.dot_general(
            hact, wd_s[e], (((1,), (1,)), ((), ())),
            preferred_element_type=jnp.float32)


def _disp_body(xpk, d3, xg, rows_v, i0, i1, s0, s1):
    wid = lax.axis_index("s") * 2 + lax.axis_index("c")
    base = wid * TPW
    pltpu.sync_copy(xpk.at[pl.ds(base, TPW)], rows_v)
    pltpu.sync_copy(d3.at[2 * wid], i0)
    pltpu.sync_copy(d3.at[2 * wid + 1], i1)
    c0 = pltpu.async_copy(rows_v, xg.at[i0], s0)
    c1 = pltpu.async_copy(rows_v, xg.at[i1], s1)
    c0.wait()
    c1.wait()


def _comb_body(yp, d3, w16, out, i0, i1, w0, w1,
               y0a, y1a, y0b, y1b, oba, obb,
               sa0, sa1, sb0, sb1, sta, stb):
    wid = lax.axis_index("s") * 2 + lax.axis_index("c")
    base = wid * TPW
    pltpu.sync_copy(d3.at[2 * wid], i0)
    pltpu.sync_copy(d3.at[2 * wid + 1], i1)
    pltpu.sync_copy(w16.at[2 * wid], w0)
    pltpu.sync_copy(w16.at[2 * wid + 1], w1)

    NCH = TPW // CH

    def fire(c, yy0, yy1, ss0, ss1):
        pltpu.async_copy(yp.at[i0.at[pl.ds(c * CH, CH)]], yy0, ss0)
        pltpu.async_copy(yp.at[i1.at[pl.ds(c * CH, CH)]], yy1, ss1)

    def wait(yy0, yy1, ss0, ss1):
        pltpu.make_async_copy(yp.at[i0.at[pl.ds(0, CH)]], yy0, ss0).wait()
        pltpu.make_async_copy(yp.at[i1.at[pl.ds(0, CH)]], yy1, ss1).wait()

    def compute(c, yy0, yy1, dst):
        for j in range(CH):
            w0j = w0[pl.ds((c * CH + j) * 16, 16)]
            w1j = w1[pl.ds((c * CH + j) * 16, 16)]
            for k in range(H // 16):
                a = yy0[j, pl.ds(k * 16, 16)]
                b = yy1[j, pl.ds(k * 16, 16)]
                dst[j, pl.ds(k * 16, 16)] = w0j * a + w1j * b

    fire(0, y0a, y1a, sa0, sa1)

    def body(k, _):
        c0 = 2 * k
        c1 = 2 * k + 1
        fire(c1, y0b, y1b, sb0, sb1)
        wait(y0a, y1a, sa0, sa1)
        compute(c0, y0a, y1a, oba)
        pltpu.async_copy(oba, out.at[pl.ds(base + c0 * CH, CH)], sta)
        cn = jnp.where(c0 + 2 < NCH, c0 + 2, 0)
        fire(cn, y0a, y1a, sa0, sa1)
        wait(y0b, y1b, sb0, sb1)
        compute(c1, y0b, y1b, obb)
        pltpu.async_copy(obb, out.at[pl.ds(base + c1 * CH, CH)], stb)
        pltpu.make_async_copy(oba, out.at[pl.ds(base, CH)], sta).wait()
        pltpu.make_async_copy(obb, out.at[pl.ds(base, CH)], stb).wait()
        return ()

    lax.fori_loop(0, NCH // 2, body, (), unroll=False)
    wait(y0a, y1a, sa0, sa1)


@functools.lru_cache(maxsize=1)
def _sc_kernels():
    mesh = plsc.VectorSubcoreMesh(core_axis_name="c", subcore_axis_name="s",
                                  num_cores=2, num_subcores=16)
    disp = pl.kernel(
        _disp_body,
        mesh=mesh,
        compiler_params=pltpu.CompilerParams(use_tc_tiling_on_sc=True),
        out_type=jax.ShapeDtypeStruct((NSLOT, H), jnp.float32),
        scratch_types=[
            pltpu.VMEM((TPW, H), jnp.float32),
            pltpu.VMEM((TPW,), jnp.int32),
            pltpu.VMEM((TPW,), jnp.int32),
            pltpu.SemaphoreType.DMA,
            pltpu.SemaphoreType.DMA,
        ],
    )
    comb = pl.kernel(
        _comb_body,
        mesh=mesh,
        compiler_params=pltpu.CompilerParams(use_tc_tiling_on_sc=True),
        out_type=jax.ShapeDtypeStruct((S, H), jnp.float32),
        scratch_types=[
            pltpu.VMEM((TPW,), jnp.int32),
            pltpu.VMEM((TPW,), jnp.int32),
            pltpu.VMEM((TPW * 16,), jnp.float32),
            pltpu.VMEM((TPW * 16,), jnp.float32),
            pltpu.VMEM((CH, H), jnp.float32),
            pltpu.VMEM((CH, H), jnp.float32),
            pltpu.VMEM((CH, H), jnp.float32),
            pltpu.VMEM((CH, H), jnp.float32),
            pltpu.VMEM((CH, H), jnp.float32),
            pltpu.VMEM((CH, H), jnp.float32),
            pltpu.SemaphoreType.DMA,
            pltpu.SemaphoreType.DMA,
            pltpu.SemaphoreType.DMA,
            pltpu.SemaphoreType.DMA,
            pltpu.SemaphoreType.DMA,
            pltpu.SemaphoreType.DMA,
        ],
    )
    return disp, comb


@jax.jit
def _run(x, gate_weight, bias2d, gate_proj, up_proj, down_proj):
    dsel, wsel, eid, nuse = pl.pallas_call(
        _gate_dispatch_body,
        out_shape=(
            jax.ShapeDtypeStruct((S, TOPK), jnp.int32),
            jax.ShapeDtypeStruct((S, TOPK), jnp.float32),
            jax.ShapeDtypeStruct((1, NT2), jnp.int32),
            jax.ShapeDtypeStruct((1, 1), jnp.int32),
        ),
    )(x, gate_weight, bias2d)

    d3 = dsel.reshape(NW, TPW, TOPK).transpose(0, 2, 1).reshape(
        NW * TOPK, TPW)
    w3 = wsel.reshape(NW, TPW, TOPK).transpose(0, 2, 1).reshape(
        NW * TOPK, TPW)

    disp, comb = _sc_kernels()
    xg = disp(x, d3)

    grid_spec = pltpu.PrefetchScalarGridSpec(
        num_scalar_prefetch=2,
        grid=(E + NT2,),
        in_specs=[
            pl.BlockSpec(
                (TM, H),
                lambda s, eid, nuse: (
                    jnp.minimum(jnp.maximum(s - E, 0), nuse[0] - 1), 0)),
            pl.BlockSpec((1, I, H),
                         lambda s, eid, nuse: (jnp.minimum(s, E - 1), 0, 0)),
            pl.BlockSpec((1, I, H),
                         lambda s, eid, nuse: (jnp.minimum(s, E - 1), 0, 0)),
            pl.BlockSpec((1, H, I),
                         lambda s, eid, nuse: (jnp.minimum(s, E - 1), 0, 0)),
        ],
        out_specs=pl.BlockSpec(
            (TM, H),
            lambda s, eid, nuse: (
                jnp.minimum(jnp.maximum(s - E, 0), nuse[0] - 1), 0)),
        scratch_shapes=[
            pltpu.VMEM((E, I, H), jnp.bfloat16),
            pltpu.VMEM((E, I, H), jnp.bfloat16),
            pltpu.VMEM((E, H, I), jnp.bfloat16),
        ],
    )
    yp = pl.pallas_call(
        _gmm_body,
        grid_spec=grid_spec,
        out_shape=jax.ShapeDtypeStruct((NSLOT, H), jnp.float32),
    )(eid.reshape(NT2), nuse.reshape(1), xg,
      gate_proj, up_proj, down_proj)

    w16 = jnp.broadcast_to(w3[:, :, None], (NW * TOPK, TPW, 16)).reshape(
        NW * TOPK, TPW * 16)
    return comb(yp, d3, w16)


def kernel(hidden_states, gate_weight, e_score_correction_bias,
           gate_proj, up_proj, down_proj):
    x = hidden_states.reshape(-1, H).astype(jnp.float32)
    bias2d = e_score_correction_bias.reshape(1, E).astype(jnp.float32)
    out = _run(x, gate_weight, bias2d, gate_proj, up_proj, down_proj)
    return out.reshape(hidden_states.shape)


# speedup vs baseline: 1.4806x; 1.2830x over previous
import jax
import jax.numpy as jnp
from jax import lax
from jax.experimental import pallas as pl
from jax.experimental.pallas import tpu as pltpu

B, S, H = 1, 2048, 1024
I = 512
E = 8
NGROUP = 4
GSIZE = E // NGROUP
SCALE = 1.0

TS = 256
NT = S // TS


def _gate(x, gw, bias):
    logits = jax.lax.dot_general(
        x, gw, (((1,), (1,)), ((), ())),
        preferred_element_type=jnp.float32)
    scores = jax.nn.sigmoid(logits)
    s4c = scores + bias

    gcols = [s4c[:, 2 * g:2 * g + 1] + s4c[:, 2 * g + 1:2 * g + 2]
             for g in range(NGROUP)]
    gexp = jnp.concatenate(
        [gcols[g] for g in range(NGROUP) for _ in range(GSIZE)], axis=1)

    eids = lax.broadcasted_iota(jnp.int32, (1, E), 1)
    gids = eids // GSIZE

    cnt = jnp.zeros((S, E), jnp.int32)
    for gp in range(NGROUP):
        gsp = gcols[gp]
        beats = (gsp > gexp) | ((gsp == gexp) & (gp < gids))
        cnt = cnt + beats.astype(jnp.int32)
    gmask = cnt < 2

    tmp = jnp.where(gmask, s4c, 0.0)
    cnt2 = jnp.zeros((S, E), jnp.int32)
    for ep in range(E):
        v = tmp[:, ep:ep + 1]
        beats = (v > tmp) | ((v == tmp) & (ep < eids))
        cnt2 = cnt2 + beats.astype(jnp.int32)
    sel = cnt2 < 2

    w = jnp.where(sel, scores, 0.0)
    denom = jnp.sum(w, axis=1, keepdims=True) + 1e-20
    return w / denom * SCALE


def _fused_body(x_ref, gw_ref, b_ref, gp_ref, up_ref, dp_ref, out_ref,
                wg_s, wu_s, wd_s, comb_s):
    s = pl.program_id(0)

    @pl.when(s < E)
    def _cast():
        wg_s[pl.ds(s, 1)] = gp_ref[...].astype(jnp.bfloat16)
        wu_s[pl.ds(s, 1)] = up_ref[...].astype(jnp.bfloat16)
        wd_s[pl.ds(s, 1)] = dp_ref[...].astype(jnp.bfloat16)

    @pl.when(s == 0)
    def _gate_step():
        comb_s[...] = _gate(x_ref[...], gw_ref[...], b_ref[...])

    @pl.when(s >= E)
    def _compute():
        t = s - E
        row = pl.ds(t * TS, TS)
        x = x_ref[row, :].astype(jnp.bfloat16)
        comb = comb_s[row, :]
        acc = jnp.zeros((TS, H), jnp.float32)
        lane = lax.broadcasted_iota(jnp.int32, (1, E), 1)
        for e in range(E):
            w = jnp.sum(jnp.where(lane == e, comb, 0.0), axis=1,
                        keepdims=True)
            g = jax.lax.dot_general(x, wg_s[e], (((1,), (1,)), ((), ())),
                                    preferred_element_type=jnp.float32)
            u = jax.lax.dot_general(x, wu_s[e], (((1,), (1,)), ((), ())),
                                    preferred_element_type=jnp.float32)
            hact = (g * jax.nn.sigmoid(g) * u).astype(jnp.bfloat16)
            y = jax.lax.dot_general(hact, wd_s[e], (((1,), (1,)), ((), ())),
                                    preferred_element_type=jnp.float32)
            acc = acc + w * y
        out_ref[...] = acc


@jax.jit
def _run(x, gate_weight, bias2d, gate_proj, up_proj, down_proj):
    out = pl.pallas_call(
        _fused_body,
        grid=(E + NT,),
        in_specs=[
            pl.BlockSpec((S, H), lambda s: (0, 0)),
            pl.BlockSpec((E, H), lambda s: (0, 0)),
            pl.BlockSpec((1, E), lambda s: (0, 0)),
            pl.BlockSpec((1, I, H), lambda s: (jnp.minimum(s, E - 1), 0, 0)),
            pl.BlockSpec((1, I, H), lambda s: (jnp.minimum(s, E - 1), 0, 0)),
            pl.BlockSpec((1, H, I), lambda s: (jnp.minimum(s, E - 1), 0, 0)),
        ],
        out_specs=pl.BlockSpec((TS, H), lambda s: (jnp.maximum(s - E, 0), 0)),
        out_shape=jax.ShapeDtypeStruct((S, H), jnp.float32),
        scratch_shapes=[
            pltpu.VMEM((E, I, H), jnp.bfloat16),
            pltpu.VMEM((E, I, H), jnp.bfloat16),
            pltpu.VMEM((E, H, I), jnp.bfloat16),
            pltpu.VMEM((S, E), jnp.float32),
        ],
    )(x, gate_weight, bias2d, gate_proj, up_proj, down_proj)
    return out


def kernel(hidden_states, gate_weight, e_score_correction_bias,
           gate_proj, up_proj, down_proj):
    x = hidden_states.reshape(-1, H).astype(jnp.float32)
    bias2d = e_score_correction_bias.reshape(1, E).astype(jnp.float32)
    out = _run(x, gate_weight, bias2d, gate_proj, up_proj, down_proj)
    return out.reshape(hidden_states.shape)


# speedup vs baseline: 1.5553x; 1.0505x over previous
import jax
import jax.numpy as jnp
from jax import lax
from jax.experimental import pallas as pl
from jax.experimental.pallas import tpu as pltpu

B, S, H = 1, 2048, 1024
I = 512
E = 8
NGROUP = 4
GSIZE = E // NGROUP
SCALE = 1.0

TS = 512
NT = S // TS


def _gate(x, gw, bias):
    logits = jax.lax.dot_general(
        x, gw, (((1,), (1,)), ((), ())),
        preferred_element_type=jnp.float32)
    scores = jax.nn.sigmoid(logits)
    s4c = scores + bias

    gcols = [s4c[:, 2 * g:2 * g + 1] + s4c[:, 2 * g + 1:2 * g + 2]
             for g in range(NGROUP)]
    gexp = jnp.concatenate(
        [gcols[g] for g in range(NGROUP) for _ in range(GSIZE)], axis=1)

    eids = lax.broadcasted_iota(jnp.int32, (1, E), 1)
    gids = eids // GSIZE

    cnt = jnp.zeros((S, E), jnp.int32)
    for gp in range(NGROUP):
        gsp = gcols[gp]
        beats = (gsp > gexp) | ((gsp == gexp) & (gp < gids))
        cnt = cnt + beats.astype(jnp.int32)
    gmask = cnt < 2

    tmp = jnp.where(gmask, s4c, 0.0)
    cnt2 = jnp.zeros((S, E), jnp.int32)
    for ep in range(E):
        v = tmp[:, ep:ep + 1]
        beats = (v > tmp) | ((v == tmp) & (ep < eids))
        cnt2 = cnt2 + beats.astype(jnp.int32)
    sel = cnt2 < 2

    w = jnp.where(sel, scores, 0.0)
    denom = jnp.sum(w, axis=1, keepdims=True) + 1e-20
    return w / denom * SCALE


def _fused_body(x_ref, gw_ref, b_ref, gp_ref, up_ref, dp_ref, out_ref,
                wg_s, wu_s, wd_s, comb_s):
    s = pl.program_id(0)

    @pl.when(s < E)
    def _cast():
        wg_s[pl.ds(s, 1)] = gp_ref[...].astype(jnp.bfloat16)
        wu_s[pl.ds(s, 1)] = up_ref[...].astype(jnp.bfloat16)
        wd_s[pl.ds(s, 1)] = dp_ref[...].astype(jnp.bfloat16)

    @pl.when(s == 0)
    def _gate_step():
        comb_s[...] = _gate(x_ref[...], gw_ref[...], b_ref[...])

    @pl.when(s >= E)
    def _compute():
        t = s - E
        row = pl.ds(t * TS, TS)
        x = x_ref[row, :].astype(jnp.bfloat16)
        comb = comb_s[row, :]
        acc = jnp.zeros((TS, H), jnp.float32)
        lane = lax.broadcasted_iota(jnp.int32, (1, E), 1)
        for e in range(E):
            w = jnp.sum(jnp.where(lane == e, comb, 0.0), axis=1,
                        keepdims=True)
            g = jax.lax.dot_general(x, wg_s[e], (((1,), (1,)), ((), ())),
                                    preferred_element_type=jnp.float32)
            u = jax.lax.dot_general(x, wu_s[e], (((1,), (1,)), ((), ())),
                                    preferred_element_type=jnp.float32)
            hact = (g * jax.nn.sigmoid(g) * u).astype(jnp.bfloat16)
            y = jax.lax.dot_general(hact, wd_s[e], (((1,), (1,)), ((), ())),
                                    preferred_element_type=jnp.float32)
            acc = acc + w * y
        out_ref[...] = acc


@jax.jit
def _run(x, gate_weight, bias2d, gate_proj, up_proj, down_proj):
    out = pl.pallas_call(
        _fused_body,
        grid=(E + NT,),
        in_specs=[
            pl.BlockSpec((S, H), lambda s: (0, 0)),
            pl.BlockSpec((E, H), lambda s: (0, 0)),
            pl.BlockSpec((1, E), lambda s: (0, 0)),
            pl.BlockSpec((1, I, H), lambda s: (jnp.minimum(s, E - 1), 0, 0)),
            pl.BlockSpec((1, I, H), lambda s: (jnp.minimum(s, E - 1), 0, 0)),
            pl.BlockSpec((1, H, I), lambda s: (jnp.minimum(s, E - 1), 0, 0)),
        ],
        out_specs=pl.BlockSpec((TS, H), lambda s: (jnp.maximum(s - E, 0), 0)),
        out_shape=jax.ShapeDtypeStruct((S, H), jnp.float32),
        scratch_shapes=[
            pltpu.VMEM((E, I, H), jnp.bfloat16),
            pltpu.VMEM((E, I, H), jnp.bfloat16),
            pltpu.VMEM((E, H, I), jnp.bfloat16),
            pltpu.VMEM((S, E), jnp.float32),
        ],
    )(x, gate_weight, bias2d, gate_proj, up_proj, down_proj)
    return out


def kernel(hidden_states, gate_weight, e_score_correction_bias,
           gate_proj, up_proj, down_proj):
    x = hidden_states.reshape(-1, H).astype(jnp.float32)
    bias2d = e_score_correction_bias.reshape(1, E).astype(jnp.float32)
    out = _run(x, gate_weight, bias2d, gate_proj, up_proj, down_proj)
    return out.reshape(hidden_states.shape)
